# baseline (speedup 1.0000x reference)
import functools

import jax
import jax.numpy as jnp
from jax import lax
from jax.experimental import pallas as pl
from jax.experimental.pallas import tpu as pltpu
from jax.experimental.pallas import tpu_sc as plsc

_NEG_INF = float("-inf")
_BIG_I32 = 2**30
_TOPK = 10
_W = 128
_KBLK = 4096
_GPB = _KBLK // _W


def _a_body(vf_ref, txt_ref, vwt_ref, vb_ref,
            logits_ref, g3_ref, m_out_ref, s_out_ref,
            img_ref, m_ref, s_ref,
            *, num_blocks, k_valid):
    step = pl.program_id(0)
    B = vf_ref.shape[0]

    @pl.when(step == 0)
    def _init():
        img = lax.dot_general(
            vf_ref[...], vwt_ref[...], (((1,), (1,)), ((), ())),
            preferred_element_type=jnp.float32) + vb_ref[...]
        nrm = jnp.sqrt(jnp.sum(img * img, axis=1, keepdims=True))
        img_ref[...] = img / nrm
        m_ref[...] = jnp.full(m_ref.shape, _NEG_INF, jnp.float32)
        s_ref[...] = jnp.zeros(s_ref.shape, jnp.float32)

    t = txt_ref[...]
    nrm = jnp.sqrt(jnp.sum(t * t, axis=1, keepdims=True))
    tn = t / nrm
    lg = 100.0 * lax.dot_general(
        img_ref[...], tn, (((1,), (1,)), ((), ())),
        preferred_element_type=jnp.float32)
    gcol = (jax.lax.broadcasted_iota(jnp.int32, (B, _KBLK), 1)
            + step * _KBLK)
    lg = jnp.where(gcol < k_valid, lg, _NEG_INF)

    gm = []
    for j in range(_GPB):
        sl = lg[:, j * _W:(j + 1) * _W]
        logits_ref[:, j, :] = sl
        gm.append(jnp.max(sl, axis=1, keepdims=True))
    gmax = jnp.concatenate(gm, axis=1)
    g3_ref[...] = gmax[None]

    bm = jnp.max(gmax, axis=1, keepdims=True)
    m_new = jnp.maximum(m_ref[...], bm)
    bs = jnp.sum(jnp.exp(lg - m_new), axis=1, keepdims=True)
    s_ref[...] = s_ref[...] * jnp.exp(m_ref[...] - m_new) + bs
    m_ref[...] = m_new

    @pl.when(step == num_blocks - 1)
    def _finalize():
        m_out_ref[...] = m_ref[...]
        s_out_ref[...] = s_ref[...]


def _b_body(g3_ref, gid_ref, lidx_ref, *, num_blocks):
    wv = jnp.concatenate([g3_ref[i] for i in range(num_blocks)], axis=1)
    B = wv.shape[0]
    G = num_blocks * _GPB
    wi = jax.lax.broadcasted_iota(jnp.int32, (B, G), 1)
    row = jax.lax.broadcasted_iota(jnp.int32, (B, 1), 0)
    for tpos in range(_TOPK):
        mx = jnp.max(wv, axis=1, keepdims=True)
        ci = jnp.min(jnp.where(wv == mx, wi, _BIG_I32),
                     axis=1, keepdims=True)
        gid_ref[:, tpos:tpos + 1] = ci
        lidx_ref[:, tpos:tpos + 1] = row * G + ci
        wv = jnp.where(wi == ci, _NEG_INF, wv)


def _d_body(cand_ref, cidx_ref, ccid_ref, m_ref, s_ref,
            out_v_ref, out_i_ref, out_a_ref):
    wv = cand_ref[...]
    wi = cidx_ref[...]
    wa = ccid_ref[...]
    inv_s = 1.0 / s_ref[...]
    for t in range(_TOPK):
        m = jnp.max(wv, axis=1, keepdims=True)
        ci = jnp.min(jnp.where(wv == m, wi, _BIG_I32), axis=1, keepdims=True)
        eqi = wi == ci
        av = jnp.sum(jnp.where(eqi, wa, 0), axis=1, keepdims=True)
        out_v_ref[:, t:t + 1] = jnp.exp(m - m_ref[...]) * inv_s
        out_i_ref[:, t:t + 1] = ci
        out_a_ref[:, t:t + 1] = av
        wv = jnp.where(eqi, _NEG_INF, wv)


def _make_sc_gather(n_idx, n_workers, width):
    b_per_w = n_idx // n_workers
    mesh = plsc.VectorSubcoreMesh(core_axis_name="c", subcore_axis_name="s")

    @functools.partial(
        pl.kernel, mesh=mesh,
        out_type=[
            jax.ShapeDtypeStruct((n_idx, width), jnp.float32),
            jax.ShapeDtypeStruct((n_idx, width), jnp.int32),
        ],
        scratch_types=[
            pltpu.VMEM((b_per_w,), jnp.int32),
            pltpu.VMEM((b_per_w,), jnp.int32),
            pltpu.VMEM((b_per_w, width), jnp.float32),
            pltpu.VMEM((b_per_w, width), jnp.int32),
            pltpu.SemaphoreType.DMA,
            pltpu.SemaphoreType.DMA,
        ],
    )
    def sc_gather(logtab, cidtab, lidx, gidx, out_log, out_cid,
                  lidx_v, gidx_v, rows_v, crows_v, sem1, sem2):
        wid = lax.axis_index("s") * 2 + lax.axis_index("c")
        base = wid * b_per_w
        pltpu.sync_copy(lidx.at[pl.ds(base, b_per_w)], lidx_v)
        pltpu.sync_copy(gidx.at[pl.ds(base, b_per_w)], gidx_v)
        cp1 = pltpu.async_copy(logtab.at[lidx_v], rows_v, sem1)
        cp2 = pltpu.async_copy(cidtab.at[gidx_v], crows_v, sem2)
        cp1.wait()
        cp2.wait()
        pltpu.sync_copy(rows_v, out_log.at[pl.ds(base, b_per_w)])
        pltpu.sync_copy(crows_v, out_cid.at[pl.ds(base, b_per_w)])

    return sc_gather


@jax.jit
def kernel(vfeats, text_features, concept_ids, v_w, v_b):
    B, D = vfeats.shape
    K = text_features.shape[0]
    k_pad = -(-K // _KBLK) * _KBLK
    num_blocks = k_pad // _KBLK
    G = k_pad // _W

    cidp = jnp.pad(concept_ids, (0, k_pad - K))
    vb2 = v_b.reshape(1, D)

    a_body = functools.partial(_a_body, num_blocks=num_blocks, k_valid=K)
    logits, g3, m_row, s_row = pl.pallas_call(
        a_body,
        grid=(num_blocks,),
        in_specs=[
            pl.BlockSpec((B, D), lambda i: (0, 0)),
            pl.BlockSpec((_KBLK, D), lambda i: (i, 0)),
            pl.BlockSpec((D, D), lambda i: (0, 0)),
            pl.BlockSpec((1, D), lambda i: (0, 0)),
        ],
        out_specs=[
            pl.BlockSpec((B, _GPB, _W), lambda i: (0, i, 0)),
            pl.BlockSpec((1, B, _GPB), lambda i: (i, 0, 0)),
            pl.BlockSpec((B, 1), lambda i: (0, 0)),
            pl.BlockSpec((B, 1), lambda i: (0, 0)),
        ],
        out_shape=[
            jax.ShapeDtypeStruct((B, G, _W), jnp.float32),
            jax.ShapeDtypeStruct((num_blocks, B, _GPB), jnp.float32),
            jax.ShapeDtypeStruct((B, 1), jnp.float32),
            jax.ShapeDtypeStruct((B, 1), jnp.float32),
        ],
        scratch_shapes=[
            pltpu.VMEM((B, D), jnp.float32),
            pltpu.VMEM((B, 1), jnp.float32),
            pltpu.VMEM((B, 1), jnp.float32),
        ],
        compiler_params=pltpu.CompilerParams(
            dimension_semantics=("arbitrary",),
        ),
    )(vfeats, text_features, v_w, vb2)

    b_body = functools.partial(_b_body, num_blocks=num_blocks)
    gid, lidx2 = pl.pallas_call(
        b_body,
        out_shape=[
            jax.ShapeDtypeStruct((B, _TOPK), jnp.int32),
            jax.ShapeDtypeStruct((B, _TOPK), jnp.int32),
        ],
    )(g3)

    n_idx = B * _TOPK
    lidx = lidx2.reshape(n_idx)
    gidx = gid.reshape(n_idx)
    logtab = logits.reshape(B * G, _W)
    cidtab = cidp.reshape(G, _W)
    glog, gcid = _make_sc_gather(n_idx, 32, _W)(logtab, cidtab, lidx, gidx)

    cand = glog.reshape(B, _TOPK * _W)
    ccid = gcid.reshape(B, _TOPK * _W)
    cidx = (gid[:, :, None] * _W
            + jnp.arange(_W, dtype=jnp.int32)[None, None, :]
            ).reshape(B, _TOPK * _W)

    values, indices, attr_ids = pl.pallas_call(
        _d_body,
        out_shape=[
            jax.ShapeDtypeStruct((B, _TOPK), jnp.float32),
            jax.ShapeDtypeStruct((B, _TOPK), jnp.int32),
            jax.ShapeDtypeStruct((B, _TOPK), jnp.int32),
        ],
    )(cand, cidx, ccid, m_row, s_row)
    return values, indices, attr_ids


# speedup vs baseline: 1.1112x; 1.1112x over previous
import functools

import jax
import jax.numpy as jnp
from jax import lax
from jax.experimental import pallas as pl
from jax.experimental.pallas import tpu as pltpu
from jax.experimental.pallas import tpu_sc as plsc

_NEG_INF = float("-inf")
_BIG_I32 = 2**30
_TOPK = 10
_W = 128
_KBLK = 4096
_GPB = _KBLK // _W


def _a_body(vf_ref, txt_ref, vwt_ref, vb_ref,
            logits_ref, g3_ref, m_out_ref, s_out_ref,
            img_ref, m_ref, s_ref,
            *, num_blocks, k_valid):
    step = pl.program_id(0)
    B = vf_ref.shape[0]

    @pl.when(step == 0)
    def _init():
        img = lax.dot_general(
            vf_ref[...], vwt_ref[...], (((1,), (1,)), ((), ())),
            preferred_element_type=jnp.float32) + vb_ref[...]
        nrm = jnp.sqrt(jnp.sum(img * img, axis=1, keepdims=True))
        img_ref[...] = img / nrm
        m_ref[...] = jnp.full(m_ref.shape, _NEG_INF, jnp.float32)
        s_ref[...] = jnp.zeros(s_ref.shape, jnp.float32)

    t = txt_ref[...]
    nrm = jnp.sqrt(jnp.sum(t * t, axis=1, keepdims=True))
    tn = t / nrm
    lg = 100.0 * lax.dot_general(
        img_ref[...], tn, (((1,), (1,)), ((), ())),
        preferred_element_type=jnp.float32)
    gcol = (jax.lax.broadcasted_iota(jnp.int32, (B, _KBLK), 1)
            + step * _KBLK)
    lg = jnp.where(gcol < k_valid, lg, _NEG_INF)

    gm = []
    for j in range(_GPB):
        sl = lg[:, j * _W:(j + 1) * _W]
        logits_ref[:, j, :, :] = sl.reshape(B // 8, 8, _W)
        gm.append(jnp.max(sl, axis=1, keepdims=True))
    gmax = jnp.concatenate(gm, axis=1)
    g3_ref[...] = gmax[None]

    bm = jnp.max(gmax, axis=1, keepdims=True)
    m_new = jnp.maximum(m_ref[...], bm)
    bs = jnp.sum(jnp.exp(lg - m_new), axis=1, keepdims=True)
    s_ref[...] = s_ref[...] * jnp.exp(m_ref[...] - m_new) + bs
    m_ref[...] = m_new

    @pl.when(step == num_blocks - 1)
    def _finalize():
        m_out_ref[...] = m_ref[...]
        s_out_ref[...] = s_ref[...]


def _b_body(g3_ref, gid_ref, lidx_ref, *, num_blocks):
    wv = jnp.concatenate([g3_ref[i] for i in range(num_blocks)], axis=1)
    B = wv.shape[0]
    G = num_blocks * _GPB
    wi = jax.lax.broadcasted_iota(jnp.int32, (B, G), 1)
    row = jax.lax.broadcasted_iota(jnp.int32, (B, 1), 0)
    tilebase = (row // 8) * (8 * G) + (row % 8)
    for tpos in range(_TOPK):
        mx = jnp.max(wv, axis=1, keepdims=True)
        ci = jnp.min(jnp.where(wv == mx, wi, _BIG_I32),
                     axis=1, keepdims=True)
        gid_ref[:, tpos:tpos + 1] = ci
        lidx_ref[:, tpos:tpos + 1] = tilebase + ci * 8
        wv = jnp.where(wi == ci, _NEG_INF, wv)


def _d_body(cand_ref, cidx_ref, ccid_ref, m_ref, s_ref,
            out_v_ref, out_i_ref, out_a_ref):
    wv = cand_ref[...]
    wi = cidx_ref[...]
    wa = ccid_ref[...]
    inv_s = 1.0 / s_ref[...]
    for t in range(_TOPK):
        m = jnp.max(wv, axis=1, keepdims=True)
        ci = jnp.min(jnp.where(wv == m, wi, _BIG_I32), axis=1, keepdims=True)
        eqi = wi == ci
        av = jnp.sum(jnp.where(eqi, wa, 0), axis=1, keepdims=True)
        out_v_ref[:, t:t + 1] = jnp.exp(m - m_ref[...]) * inv_s
        out_i_ref[:, t:t + 1] = ci
        out_a_ref[:, t:t + 1] = av
        wv = jnp.where(eqi, _NEG_INF, wv)


def _make_sc_gather(n_idx, n_workers, width):
    b_per_w = n_idx // n_workers
    mesh = plsc.VectorSubcoreMesh(core_axis_name="c", subcore_axis_name="s")

    @functools.partial(
        pl.kernel, mesh=mesh,
        out_type=[
            jax.ShapeDtypeStruct((n_idx, width), jnp.float32),
            jax.ShapeDtypeStruct((n_idx, width), jnp.int32),
        ],
        scratch_types=[
            pltpu.VMEM((b_per_w,), jnp.int32),
            pltpu.VMEM((b_per_w,), jnp.int32),
            pltpu.VMEM((b_per_w, width), jnp.float32),
            pltpu.VMEM((b_per_w, width), jnp.int32),
            pltpu.SemaphoreType.DMA,
            pltpu.SemaphoreType.DMA,
        ],
    )
    def sc_gather(logtab, cidtab, lidx, gidx, out_log, out_cid,
                  lidx_v, gidx_v, rows_v, crows_v, sem1, sem2):
        wid = lax.axis_index("s") * 2 + lax.axis_index("c")
        base = wid * b_per_w
        pltpu.sync_copy(lidx.at[pl.ds(base, b_per_w)], lidx_v)
        pltpu.sync_copy(gidx.at[pl.ds(base, b_per_w)], gidx_v)
        cp1 = pltpu.async_copy(logtab.at[lidx_v], rows_v, sem1)
        cp2 = pltpu.async_copy(cidtab.at[gidx_v], crows_v, sem2)
        cp1.wait()
        cp2.wait()
        pltpu.sync_copy(rows_v, out_log.at[pl.ds(base, b_per_w)])
        pltpu.sync_copy(crows_v, out_cid.at[pl.ds(base, b_per_w)])

    return sc_gather


@jax.jit
def kernel(vfeats, text_features, concept_ids, v_w, v_b):
    B, D = vfeats.shape
    K = text_features.shape[0]
    k_pad = -(-K // _KBLK) * _KBLK
    num_blocks = k_pad // _KBLK
    G = k_pad // _W

    cidp = jnp.pad(concept_ids, (0, k_pad - K))
    vb2 = v_b.reshape(1, D)

    a_body = functools.partial(_a_body, num_blocks=num_blocks, k_valid=K)
    logits, g3, m_row, s_row = pl.pallas_call(
        a_body,
        grid=(num_blocks,),
        in_specs=[
            pl.BlockSpec((B, D), lambda i: (0, 0)),
            pl.BlockSpec((_KBLK, D), lambda i: (i, 0)),
            pl.BlockSpec((D, D), lambda i: (0, 0)),
            pl.BlockSpec((1, D), lambda i: (0, 0)),
        ],
        out_specs=[
            pl.BlockSpec((B // 8, _GPB, 8, _W), lambda i: (0, i, 0, 0)),
            pl.BlockSpec((1, B, _GPB), lambda i: (i, 0, 0)),
            pl.BlockSpec((B, 1), lambda i: (0, 0)),
            pl.BlockSpec((B, 1), lambda i: (0, 0)),
        ],
        out_shape=[
            jax.ShapeDtypeStruct((B // 8, G, 8, _W), jnp.float32),
            jax.ShapeDtypeStruct((num_blocks, B, _GPB), jnp.float32),
            jax.ShapeDtypeStruct((B, 1), jnp.float32),
            jax.ShapeDtypeStruct((B, 1), jnp.float32),
        ],
        scratch_shapes=[
            pltpu.VMEM((B, D), jnp.float32),
            pltpu.VMEM((B, 1), jnp.float32),
            pltpu.VMEM((B, 1), jnp.float32),
        ],
        compiler_params=pltpu.CompilerParams(
            dimension_semantics=("arbitrary",),
        ),
    )(vfeats, text_features, v_w, vb2)

    b_body = functools.partial(_b_body, num_blocks=num_blocks)
    gid, lidx2 = pl.pallas_call(
        b_body,
        out_shape=[
            jax.ShapeDtypeStruct((B, _TOPK), jnp.int32),
            jax.ShapeDtypeStruct((B, _TOPK), jnp.int32),
        ],
    )(g3)

    n_idx = B * _TOPK
    lidx = lidx2.reshape(n_idx)
    gidx = gid.reshape(n_idx)
    logtab = logits.reshape(B * G, _W)
    cidtab = cidp.reshape(G, _W)
    glog, gcid = _make_sc_gather(n_idx, 32, _W)(logtab, cidtab, lidx, gidx)

    cand = glog.reshape(B, _TOPK * _W)
    ccid = gcid.reshape(B, _TOPK * _W)
    cidx = (gid[:, :, None] * _W
            + jnp.arange(_W, dtype=jnp.int32)[None, None, :]
            ).reshape(B, _TOPK * _W)

    values, indices, attr_ids = pl.pallas_call(
        _d_body,
        out_shape=[
            jax.ShapeDtypeStruct((B, _TOPK), jnp.float32),
            jax.ShapeDtypeStruct((B, _TOPK), jnp.int32),
            jax.ShapeDtypeStruct((B, _TOPK), jnp.int32),
        ],
    )(cand, cidx, ccid, m_row, s_row)
    return values, indices, attr_ids


# speedup vs baseline: 1.1420x; 1.0277x over previous
import functools

import jax
import jax.numpy as jnp
from jax import lax
from jax.experimental import pallas as pl
from jax.experimental.pallas import tpu as pltpu
from jax.experimental.pallas import tpu_sc as plsc

_NEG_INF = float("-inf")
_BIG_I32 = 2**30
_TOPK = 10
_W = 128
_KBLK = 8192
_GPB = _KBLK // _W


def _a_body(vf_ref, txt_ref, vwt_ref, vb_ref,
            logits_ref, g3_ref, m_out_ref, s_out_ref,
            img_ref, m_ref, s_ref,
            *, num_blocks, k_valid):
    step = pl.program_id(0)
    B = vf_ref.shape[0]

    @pl.when(step == 0)
    def _init():
        img = lax.dot_general(
            vf_ref[...], vwt_ref[...], (((1,), (1,)), ((), ())),
            preferred_element_type=jnp.float32) + vb_ref[...]
        nrm = jnp.sqrt(jnp.sum(img * img, axis=1, keepdims=True))
        img_ref[...] = img / nrm
        m_ref[...] = jnp.full(m_ref.shape, _NEG_INF, jnp.float32)
        s_ref[...] = jnp.zeros(s_ref.shape, jnp.float32)

    t = txt_ref[...]
    nrm = jnp.sqrt(jnp.sum(t * t, axis=1, keepdims=True))
    tn = t / nrm
    lg = 100.0 * lax.dot_general(
        img_ref[...], tn, (((1,), (1,)), ((), ())),
        preferred_element_type=jnp.float32)
    gcol = (jax.lax.broadcasted_iota(jnp.int32, (B, _KBLK), 1)
            + step * _KBLK)
    lg = jnp.where(gcol < k_valid, lg, _NEG_INF)

    gm = []
    for j in range(_GPB):
        sl = lg[:, j * _W:(j + 1) * _W]
        logits_ref[:, j, :, :] = sl.reshape(B // 8, 8, _W)
        gm.append(jnp.max(sl, axis=1, keepdims=True))
    gmax = jnp.concatenate(gm, axis=1)
    g3_ref[...] = gmax[None]

    bm = jnp.max(gmax, axis=1, keepdims=True)
    m_new = jnp.maximum(m_ref[...], bm)
    bs = jnp.sum(jnp.exp(lg - m_new), axis=1, keepdims=True)
    s_ref[...] = s_ref[...] * jnp.exp(m_ref[...] - m_new) + bs
    m_ref[...] = m_new

    @pl.when(step == num_blocks - 1)
    def _finalize():
        m_out_ref[...] = m_ref[...]
        s_out_ref[...] = s_ref[...]


def _b_body(g3_ref, gid_ref, lidx_ref, *, num_blocks):
    wv = jnp.concatenate([g3_ref[i] for i in range(num_blocks)], axis=1)
    B = wv.shape[0]
    G = num_blocks * _GPB
    wi = jax.lax.broadcasted_iota(jnp.int32, (B, G), 1)
    row = jax.lax.broadcasted_iota(jnp.int32, (B, 1), 0)
    tilebase = (row // 8) * (8 * G) + (row % 8)
    for tpos in range(_TOPK):
        mx = jnp.max(wv, axis=1, keepdims=True)
        ci = jnp.min(jnp.where(wv == mx, wi, _BIG_I32),
                     axis=1, keepdims=True)
        gid_ref[:, tpos:tpos + 1] = ci
        lidx_ref[:, tpos:tpos + 1] = tilebase + ci * 8
        wv = jnp.where(wi == ci, _NEG_INF, wv)


def _d_body(cand_ref, cidx_ref, ccid_ref, m_ref, s_ref,
            out_v_ref, out_i_ref, out_a_ref):
    wv = cand_ref[...]
    wi = cidx_ref[...]
    wa = ccid_ref[...]
    inv_s = 1.0 / s_ref[...]
    for t in range(_TOPK):
        m = jnp.max(wv, axis=1, keepdims=True)
        ci = jnp.min(jnp.where(wv == m, wi, _BIG_I32), axis=1, keepdims=True)
        eqi = wi == ci
        av = jnp.sum(jnp.where(eqi, wa, 0), axis=1, keepdims=True)
        out_v_ref[:, t:t + 1] = jnp.exp(m - m_ref[...]) * inv_s
        out_i_ref[:, t:t + 1] = ci
        out_a_ref[:, t:t + 1] = av
        wv = jnp.where(eqi, _NEG_INF, wv)


def _make_sc_gather(n_idx, n_workers, width):
    b_per_w = n_idx // n_workers
    mesh = plsc.VectorSubcoreMesh(core_axis_name="c", subcore_axis_name="s")

    @functools.partial(
        pl.kernel, mesh=mesh,
        out_type=[
            jax.ShapeDtypeStruct((n_idx, width), jnp.float32),
            jax.ShapeDtypeStruct((n_idx, width), jnp.int32),
        ],
        scratch_types=[
            pltpu.VMEM((b_per_w,), jnp.int32),
            pltpu.VMEM((b_per_w,), jnp.int32),
            pltpu.VMEM((b_per_w, width), jnp.float32),
            pltpu.VMEM((b_per_w, width), jnp.int32),
            pltpu.SemaphoreType.DMA,
            pltpu.SemaphoreType.DMA,
        ],
    )
    def sc_gather(logtab, cidtab, lidx, gidx, out_log, out_cid,
                  lidx_v, gidx_v, rows_v, crows_v, sem1, sem2):
        wid = lax.axis_index("s") * 2 + lax.axis_index("c")
        base = wid * b_per_w
        pltpu.sync_copy(lidx.at[pl.ds(base, b_per_w)], lidx_v)
        pltpu.sync_copy(gidx.at[pl.ds(base, b_per_w)], gidx_v)
        cp1 = pltpu.async_copy(logtab.at[lidx_v], rows_v, sem1)
        cp2 = pltpu.async_copy(cidtab.at[gidx_v], crows_v, sem2)
        cp1.wait()
        cp2.wait()
        pltpu.sync_copy(rows_v, out_log.at[pl.ds(base, b_per_w)])
        pltpu.sync_copy(crows_v, out_cid.at[pl.ds(base, b_per_w)])

    return sc_gather


@jax.jit
def kernel(vfeats, text_features, concept_ids, v_w, v_b):
    B, D = vfeats.shape
    K = text_features.shape[0]
    k_pad = -(-K // _KBLK) * _KBLK
    num_blocks = k_pad // _KBLK
    G = k_pad // _W

    cidp = jnp.pad(concept_ids, (0, k_pad - K))
    vb2 = v_b.reshape(1, D)

    a_body = functools.partial(_a_body, num_blocks=num_blocks, k_valid=K)
    logits, g3, m_row, s_row = pl.pallas_call(
        a_body,
        grid=(num_blocks,),
        in_specs=[
            pl.BlockSpec((B, D), lambda i: (0, 0)),
            pl.BlockSpec((_KBLK, D), lambda i: (i, 0)),
            pl.BlockSpec((D, D), lambda i: (0, 0)),
            pl.BlockSpec((1, D), lambda i: (0, 0)),
        ],
        out_specs=[
            pl.BlockSpec((B // 8, _GPB, 8, _W), lambda i: (0, i, 0, 0)),
            pl.BlockSpec((1, B, _GPB), lambda i: (i, 0, 0)),
            pl.BlockSpec((B, 1), lambda i: (0, 0)),
            pl.BlockSpec((B, 1), lambda i: (0, 0)),
        ],
        out_shape=[
            jax.ShapeDtypeStruct((B // 8, G, 8, _W), jnp.float32),
            jax.ShapeDtypeStruct((num_blocks, B, _GPB), jnp.float32),
            jax.ShapeDtypeStruct((B, 1), jnp.float32),
            jax.ShapeDtypeStruct((B, 1), jnp.float32),
        ],
        scratch_shapes=[
            pltpu.VMEM((B, D), jnp.float32),
            pltpu.VMEM((B, 1), jnp.float32),
            pltpu.VMEM((B, 1), jnp.float32),
        ],
        compiler_params=pltpu.CompilerParams(
            dimension_semantics=("arbitrary",),
        ),
    )(vfeats, text_features, v_w, vb2)

    b_body = functools.partial(_b_body, num_blocks=num_blocks)
    gid, lidx2 = pl.pallas_call(
        b_body,
        out_shape=[
            jax.ShapeDtypeStruct((B, _TOPK), jnp.int32),
            jax.ShapeDtypeStruct((B, _TOPK), jnp.int32),
        ],
    )(g3)

    n_idx = B * _TOPK
    lidx = lidx2.reshape(n_idx)
    gidx = gid.reshape(n_idx)
    logtab = logits.reshape(B * G, _W)
    cidtab = cidp.reshape(G, _W)
    glog, gcid = _make_sc_gather(n_idx, 32, _W)(logtab, cidtab, lidx, gidx)

    cand = glog.reshape(B, _TOPK * _W)
    ccid = gcid.reshape(B, _TOPK * _W)
    cidx = (gid[:, :, None] * _W
            + jnp.arange(_W, dtype=jnp.int32)[None, None, :]
            ).reshape(B, _TOPK * _W)

    values, indices, attr_ids = pl.pallas_call(
        _d_body,
        out_shape=[
            jax.ShapeDtypeStruct((B, _TOPK), jnp.float32),
            jax.ShapeDtypeStruct((B, _TOPK), jnp.int32),
            jax.ShapeDtypeStruct((B, _TOPK), jnp.int32),
        ],
    )(cand, cidx, ccid, m_row, s_row)
    return values, indices, attr_ids
